# baseline (speedup 1.0000x reference)
import functools

import jax
import jax.numpy as jnp
from jax import lax
from jax.experimental import pallas as pl
from jax.experimental.pallas import tpu as pltpu
from jax.experimental.pallas import tpu_sc as plsc

VOCAB = 100000
EMB = 128
HID = 128
L = 50
B = 32

_NW = 32
_N_IDX = L * B
_BPW = 64
_NW_USED = _N_IDX // _BPW

_VT = 4096
_NV = (VOCAB + _VT - 1) // _VT


def _sc_gather(table, idx):
    mesh = plsc.VectorSubcoreMesh(core_axis_name="c", subcore_axis_name="s")

    @functools.partial(
        pl.kernel,
        mesh=mesh,
        out_type=jax.ShapeDtypeStruct((_N_IDX, EMB), jnp.float32),
        scratch_types=[
            pltpu.VMEM((_BPW,), jnp.int32),
            pltpu.VMEM((_BPW, EMB), jnp.float32),
            pltpu.SemaphoreType.DMA,
        ],
    )
    def gather_kernel(table_hbm, idx_hbm, out_hbm, idx_v, rows_v, sem):
        wid = lax.axis_index("s") * 2 + lax.axis_index("c")

        @pl.when(wid < _NW_USED)
        def _do():
            base = wid * _BPW
            pltpu.sync_copy(idx_hbm.at[pl.ds(base, _BPW)], idx_v)
            pltpu.async_copy(table_hbm.at[idx_v], rows_v, sem).wait()
            pltpu.sync_copy(rows_v, out_hbm.at[pl.ds(base, _BPW)])

    return gather_kernel(table, idx)


def _fused_body(emb_ref, wih_ref, whh_ref, bih_ref, bhh_ref,
                wout_ref, bout_ref, out_ref, hs_scr):
    @pl.when(pl.program_id(0) == 0)
    def _run_rnn():
        wih = wih_ref[...]
        whh = whh_ref[...]
        bias = bih_ref[...] + bhh_ref[...]

        def step(t, h):
            x = emb_ref[pl.ds(t * B, B), :]
            pre = (
                lax.dot_general(x, wih, (((1,), (1,)), ((), ())))
                + lax.dot_general(h, whh, (((1,), (1,)), ((), ())))
                + bias
            )
            h_new = jnp.tanh(pre)
            hs_scr[pl.ds(t * B, B), :] = h_new.astype(jnp.bfloat16)
            return h_new

        lax.fori_loop(0, L, step, jnp.zeros((B, HID), jnp.float32))

    w = wout_ref[...].astype(jnp.bfloat16)
    r = lax.dot_general(
        hs_scr[...], w,
        (((1,), (1,)), ((), ())),
        preferred_element_type=jnp.float32,
    )
    out_ref[...] = r + bout_ref[...]


def kernel(input_sequence, table, W_ih, W_hh, b_ih, b_hh, W_out, b_out):
    idx = input_sequence.reshape(-1).astype(jnp.int32)
    emb = _sc_gather(table, idx)

    out2d = pl.pallas_call(
        _fused_body,
        grid=(_NV,),
        in_specs=[
            pl.BlockSpec((_N_IDX, EMB), lambda v: (0, 0)),
            pl.BlockSpec((HID, EMB), lambda v: (0, 0)),
            pl.BlockSpec((HID, HID), lambda v: (0, 0)),
            pl.BlockSpec((1, HID), lambda v: (0, 0)),
            pl.BlockSpec((1, HID), lambda v: (0, 0)),
            pl.BlockSpec((_VT, HID), lambda v: (v, 0)),
            pl.BlockSpec((1, _VT), lambda v: (0, v)),
        ],
        out_specs=pl.BlockSpec((L * B, _VT), lambda v: (0, v)),
        out_shape=jax.ShapeDtypeStruct((L * B, VOCAB), jnp.float32),
        scratch_shapes=[pltpu.VMEM((L * B, HID), jnp.bfloat16)],
    )(emb, W_ih, W_hh, b_ih.reshape(1, HID), b_hh.reshape(1, HID),
      W_out, b_out.reshape(1, VOCAB))

    return jnp.transpose(out2d.reshape(L, B, VOCAB), (0, 2, 1))


# speedup vs baseline: 1.0065x; 1.0065x over previous
import functools

import jax
import jax.numpy as jnp
from jax import lax
from jax.experimental import pallas as pl
from jax.experimental.pallas import tpu as pltpu
from jax.experimental.pallas import tpu_sc as plsc

VOCAB = 100000
EMB = 128
HID = 128
L = 50
B = 32

_NW = 32
_N_IDX = L * B
_BPW = 64
_NW_USED = _N_IDX // _BPW

_VT = 3072
_NV = (VOCAB + _VT - 1) // _VT


def _sc_gather(table, idx):
    mesh = plsc.VectorSubcoreMesh(core_axis_name="c", subcore_axis_name="s")

    @functools.partial(
        pl.kernel,
        mesh=mesh,
        out_type=jax.ShapeDtypeStruct((_N_IDX, EMB), jnp.float32),
        scratch_types=[
            pltpu.VMEM((_BPW,), jnp.int32),
            pltpu.VMEM((_BPW, EMB), jnp.float32),
            pltpu.SemaphoreType.DMA,
        ],
    )
    def gather_kernel(table_hbm, idx_hbm, out_hbm, idx_v, rows_v, sem):
        wid = lax.axis_index("s") * 2 + lax.axis_index("c")

        @pl.when(wid < _NW_USED)
        def _do():
            base = wid * _BPW
            pltpu.sync_copy(idx_hbm.at[pl.ds(base, _BPW)], idx_v)
            pltpu.async_copy(table_hbm.at[idx_v], rows_v, sem).wait()
            pltpu.sync_copy(rows_v, out_hbm.at[pl.ds(base, _BPW)])

    return gather_kernel(table, idx)


def _fused_body(emb_ref, wih_ref, whh_ref, bih_ref, bhh_ref,
                wout_ref, bout_ref, out_ref, hs_scr):
    @pl.when(pl.program_id(0) == 0)
    def _run_rnn():
        wih = wih_ref[...]
        whh = whh_ref[...]
        bias = bih_ref[...] + bhh_ref[...]

        def step(t, h):
            x = emb_ref[pl.ds(t * B, B), :]
            pre = (
                lax.dot_general(x, wih, (((1,), (1,)), ((), ())))
                + lax.dot_general(h, whh, (((1,), (1,)), ((), ())))
                + bias
            )
            h_new = jnp.tanh(pre)
            hs_scr[pl.ds(t * B, B), :] = h_new.astype(jnp.bfloat16)
            return h_new

        lax.fori_loop(0, L, step, jnp.zeros((B, HID), jnp.float32))

    w = wout_ref[...].astype(jnp.bfloat16)
    r = lax.dot_general(
        hs_scr[...], w,
        (((1,), (1,)), ((), ())),
        preferred_element_type=jnp.float32,
    )
    out_ref[...] = r + bout_ref[...]


def kernel(input_sequence, table, W_ih, W_hh, b_ih, b_hh, W_out, b_out):
    idx = input_sequence.reshape(-1).astype(jnp.int32)
    emb = _sc_gather(table, idx)

    out2d = pl.pallas_call(
        _fused_body,
        grid=(_NV,),
        in_specs=[
            pl.BlockSpec((_N_IDX, EMB), lambda v: (0, 0)),
            pl.BlockSpec((HID, EMB), lambda v: (0, 0)),
            pl.BlockSpec((HID, HID), lambda v: (0, 0)),
            pl.BlockSpec((1, HID), lambda v: (0, 0)),
            pl.BlockSpec((1, HID), lambda v: (0, 0)),
            pl.BlockSpec((_VT, HID), lambda v: (v, 0)),
            pl.BlockSpec((1, _VT), lambda v: (0, v)),
        ],
        out_specs=pl.BlockSpec((L * B, _VT), lambda v: (0, v)),
        out_shape=jax.ShapeDtypeStruct((L * B, VOCAB), jnp.float32),
        scratch_shapes=[pltpu.VMEM((L * B, HID), jnp.bfloat16)],
    )(emb, W_ih, W_hh, b_ih.reshape(1, HID), b_hh.reshape(1, HID),
      W_out, b_out.reshape(1, VOCAB))

    return jnp.transpose(out2d.reshape(L, B, VOCAB), (0, 2, 1))
